# baseline (speedup 1.0000x reference)
import functools

import jax
import jax.numpy as jnp
from jax import lax
from jax.experimental import pallas as pl
from jax.experimental.pallas import tpu as pltpu
from jax.experimental.pallas import tpu_sc as plsc

N = 10000
E = 160000
G = 64
IN_CH = 5
HID = 512
OUT = 256

NC = 2
NS = 16
NW = NC * NS
L = 16

N_PAD = 10240
NPW = N_PAD // NW
NPT = N_PAD // NS
E_PAD = 163840
EPW = E_PAD // NW
EPT = E_PAD // NS
BE = 512
SUBC = BE // 128
MB = EPW // BE
DB = EPT // BE
AGG_W = IN_CH * N_PAD
AGG_R = AGG_W // 128
X_W = 50048
D_ROWS = G * N_PAD
D_PT = D_ROWS // NS
ZB = 2048

_mesh = plsc.VectorSubcoreMesh(core_axis_name="c", subcore_axis_name="s")
_params = pltpu.CompilerParams(needs_layout_passes=False)


@functools.partial(
    pl.kernel,
    out_type=(
        jax.ShapeDtypeStruct((N_PAD,), jnp.float32),
        jax.ShapeDtypeStruct((E_PAD,), jnp.float32),
        jax.ShapeDtypeStruct((NC * D_ROWS,), jnp.float32),
    ),
    mesh=_mesh,
    compiler_params=_params,
    scratch_types=[
        pltpu.VMEM((N_PAD,), jnp.float32),
        pltpu.VMEM((N_PAD,), jnp.float32),
        pltpu.VMEM((N_PAD,), jnp.int32),
        pltpu.VMEM((BE,), jnp.int32),
        pltpu.VMEM((BE,), jnp.int32),
        pltpu.VMEM((BE,), jnp.float32),
        pltpu.VMEM((BE,), jnp.float32),
        pltpu.VMEM((BE,), jnp.int32),
        pltpu.VMEM((BE,), jnp.int32),
        pltpu.VMEM((1, 128), jnp.int32),
        pltpu.VMEM((1, 128), jnp.int32),
        pltpu.VMEM((1, 128), jnp.int32),
        pltpu.VMEM((1, 128), jnp.int32),
        pltpu.VMEM((BE,), jnp.float32),
        pltpu.VMEM((128,), jnp.float32),
        pltpu.SemaphoreType.DMA,
        pltpu.VMEM((NPT,), jnp.float32),
        pltpu.VMEM((NPT,), jnp.float32),
        pltpu.VMEM((ZB,), jnp.float32),
        pltpu.SemaphoreType.DMA,
        pltpu.SemaphoreType.DMA,
        pltpu.VMEM_SHARED((N_PAD,), jnp.float32),
        pltpu.VMEM_SHARED((N_PAD,), jnp.int32),
        pltpu.VMEM_SHARED((D_ROWS,), jnp.float32),
        pltpu.VMEM_SHARED((NS * N_PAD,), jnp.float32),
    ],
)
def _sc_main(rowf, colf, wf, batchf, disf, normf, Dp,
             degl, disl, batchl, cb0, cb1, wb0, wb1, rb0, rb1,
             didx0, didx1, didx2, didx3, nva, nv0, sfl, accb, tmp0, zb,
             sin0, sin1,
             dis_sh, batch_sh, D_sh, slots):
    cid = lax.axis_index("c")
    sid = lax.axis_index("s")
    wid = cid * NS + sid
    iota = lax.iota(jnp.int32, L)
    z16 = jnp.zeros((L,), jnp.int32)
    zf16 = jnp.zeros((L,), jnp.float32)
    cbs, wbs, rbs = (cb0, cb1), (wb0, wb1), (rb0, rb1)
    sins = (sin0, sin1)

    @pl.loop(0, ZB // L)
    def _(q):
        zb[pl.ds(q * L, L)] = zf16

    @pl.loop(0, D_PT // ZB)
    def _(q):
        pltpu.sync_copy(zb, D_sh.at[pl.ds(sid * D_PT + q * ZB, ZB)])

    @pl.loop(0, N_PAD // L)
    def _(q):
        degl[pl.ds(q * L, L)] = zf16

    @pl.when(sid == 0)
    def _():
        pltpu.sync_copy(batchf, batch_sh)

    dbase = sid * EPT

    def _deg_start(g, t):
        off = dbase + g * BE
        pltpu.async_copy(colf.at[pl.ds(off, BE)], cbs[t], sins[t])
        pltpu.async_copy(wf.at[pl.ds(off, BE)], wbs[t], sins[t])

    def _deg_wait(g, t):
        off = dbase + g * BE
        pltpu.make_async_copy(colf.at[pl.ds(off, BE)], cbs[t], sins[t]).wait()
        pltpu.make_async_copy(wf.at[pl.ds(off, BE)], wbs[t], sins[t]).wait()

    def _deg_proc(t):
        def group(q, carry):
            p = q * L
            ci = cbs[t][pl.ds(p, L)]
            wv = wbs[t][pl.ds(p, L)]
            plsc.addupdate_scatter(degl, [ci], wv)
            return carry

        lax.fori_loop(0, BE // L, group, 0)

    _deg_start(0, 0)

    @pl.loop(0, DB, step=2)
    def _(g):
        _deg_wait(g, 0)
        _deg_start(g + 1, 1)
        _deg_proc(0)
        _deg_wait(g + 1, 1)

        @pl.when(g + 2 < DB)
        def _():
            _deg_start(g + 2, 0)

        _deg_proc(1)

    pltpu.sync_copy(degl, slots.at[pl.ds(sid * N_PAD, N_PAD)])
    plsc.subcore_barrier()
    nb0 = sid * NPT
    pltpu.sync_copy(slots.at[pl.ds(nb0, NPT)], accb)
    for k in range(1, NS):
        pltpu.sync_copy(slots.at[pl.ds(k * N_PAD + nb0, NPT)], tmp0)

        def addg(q, carry):
            sl = pl.ds(q * L, L)
            accb[sl] = accb[sl] + tmp0[sl]
            return carry

        lax.fori_loop(0, NPT // L, addg, 0)

    for gg in range(NPT // L):
        sl = pl.ds(gg * L, L)
        d = accb[sl] + 1.0
        i32 = plsc.bitcast(d, jnp.int32)
        seed = jnp.int32(0x5F3759DF) - lax.shift_right_arithmetic(i32, 1)
        y = plsc.bitcast(seed, jnp.float32)
        for _ in range(3):
            y = y * (1.5 - 0.5 * d * y * y)
        accb[sl] = y
    pltpu.sync_copy(accb, dis_sh.at[pl.ds(nb0, NPT)])

    @pl.when(cid == 0)
    def _():
        pltpu.sync_copy(accb, disf.at[pl.ds(nb0, NPT)])

    plsc.subcore_barrier()
    pltpu.sync_copy(dis_sh, disl)
    pltpu.sync_copy(batch_sh, batchl)

    ebase = wid * EPW

    def _in_start(g, t):
        off = ebase + g * BE
        pltpu.async_copy(rowf.at[pl.ds(off, BE)], rbs[t], sins[t])
        pltpu.async_copy(colf.at[pl.ds(off, BE)], cbs[t], sins[t])
        pltpu.async_copy(wf.at[pl.ds(off, BE)], wbs[t], sins[t])

    def _in_wait(g, t):
        off = ebase + g * BE
        pltpu.make_async_copy(rowf.at[pl.ds(off, BE)], rbs[t], sins[t]).wait()
        pltpu.make_async_copy(colf.at[pl.ds(off, BE)], cbs[t], sins[t]).wait()
        pltpu.make_async_copy(wf.at[pl.ds(off, BE)], wbs[t],
                              sins[t]).wait()

    didxs = (didx0, didx1, didx2, didx3)

    def _proc_burst(g, t):
        descs = []
        for sc_i in range(SUBC):
            for k in range(8):
                p = sc_i * 128 + k * L
                sl = pl.ds(p, L)
                r = rbs[t][sl]
                c = cbs[t][sl]
                wv = wbs[t][sl]
                dr = plsc.load_gather(disl, [r])
                dc = plsc.load_gather(disl, [c])
                nv = dr * wv * dc
                gb = plsc.load_gather(batchl, [c])
                di = gb * N_PAD + r
                didxs[sc_i][0, pl.ds(k * L, L)] = di
                nva[pl.ds(p, L)] = nv
            pltpu.sync_copy(nva.at[pl.ds(sc_i * 128, 128)],
                            D_sh.at[didxs[sc_i].at[0]], add=True)
        descs.append(pltpu.async_copy(
            nva, normf.at[pl.ds(ebase + g * BE, BE)], sfl))
        for dsc in descs:
            dsc.wait()

    _in_start(0, 0)

    @pl.loop(0, MB, step=2)
    def _(g):
        _in_wait(g, 0)
        _in_start(g + 1, 1)
        _proc_burst(g, 0)
        _in_wait(g + 1, 1)

        @pl.when(g + 2 < MB)
        def _():
            _in_start(g + 2, 0)

        _proc_burst(g + 1, 1)

    sbase = wid * NPW
    for t in range(3):
        for k in range(8):
            gq = t * 8 + k
            if gq < NPW // L:
                n16 = sbase + gq * L + iota
                dn = plsc.load_gather(disl, [n16])
                v2 = jnp.where(n16 < N, dn * dn, 0.0)
                gb = plsc.load_gather(batchl, [n16])
                di = gb * N_PAD + n16
                didx0[0, pl.ds(k * L, L)] = di
                nv0[pl.ds(k * L, L)] = v2
            else:
                didx0[0, pl.ds(k * L, L)] = z16
                nv0[pl.ds(k * L, L)] = zf16
        pltpu.sync_copy(nv0, D_sh.at[didx0.at[0]], add=True)

    plsc.subcore_barrier()
    pltpu.sync_copy(D_sh.at[pl.ds(sid * D_PT, D_PT)],
                    Dp.at[pl.ds(cid * D_ROWS + sid * D_PT, D_PT)])


@functools.partial(
    pl.kernel,
    out_type=jax.ShapeDtypeStruct((NC, AGG_R, 128), jnp.float32),
    mesh=_mesh,
    compiler_params=_params,
    scratch_types=[
        pltpu.VMEM((AGG_R, 128), jnp.float32),
        pltpu.VMEM((X_W,), jnp.float32),
        pltpu.VMEM((BE,), jnp.int32),
        pltpu.VMEM((BE,), jnp.int32),
        pltpu.VMEM((BE,), jnp.int32),
        pltpu.VMEM((BE,), jnp.int32),
        pltpu.VMEM((BE,), jnp.float32),
        pltpu.VMEM((BE,), jnp.float32),
        pltpu.VMEM((4, 100), jnp.int32),
        pltpu.VMEM((8, 128), jnp.float32),
        pltpu.SemaphoreType.DMA,
        pltpu.SemaphoreType.DMA,
        pltpu.VMEM_SHARED((AGG_R, 128), jnp.float32),
        pltpu.VMEM_SHARED((X_W,), jnp.float32),
    ],
)
def _sc_agg(rowf, colf, normf, xpad, ridx4, aggp,
            aggl, xl, rb0, rb1, cb0, cb1, nb0, nb1, ridxb, zb,
            sin0, sin1, agg_sh, x_sh):
    cid = lax.axis_index("c")
    sid = lax.axis_index("s")
    wid = cid * NS + sid
    zf16 = jnp.zeros((L,), jnp.float32)
    rbs, cbs, nbs = (rb0, rb1), (cb0, cb1), (nb0, nb1)
    sins = (sin0, sin1)
    pltpu.sync_copy(ridx4, ridxb)
    for rr in range(8):
        for c8 in range(8):
            zb[rr, pl.ds(c8 * L, L)] = zf16

    @pl.when(sid == 0)
    def _():
        pltpu.sync_copy(xpad, x_sh)

    for j in range(3):
        pltpu.sync_copy(zb, agg_sh.at[pl.ds((sid + 16 * j) * 8, 8)])

    @pl.when(sid < 2)
    def _():
        pltpu.sync_copy(zb, agg_sh.at[pl.ds((sid + 48) * 8, 8)])

    plsc.subcore_barrier()
    pltpu.sync_copy(x_sh, xl)
    pltpu.sync_copy(agg_sh, aggl)
    plsc.subcore_barrier()
    ebase = wid * EPW

    def _in_start(g, t):
        off = ebase + g * BE
        pltpu.async_copy(rowf.at[pl.ds(off, BE)], rbs[t], sins[t])
        pltpu.async_copy(colf.at[pl.ds(off, BE)], cbs[t], sins[t])
        pltpu.async_copy(normf.at[pl.ds(off, BE)], nbs[t], sins[t])

    def _in_wait(g, t):
        off = ebase + g * BE
        pltpu.make_async_copy(rowf.at[pl.ds(off, BE)], rbs[t], sins[t]).wait()
        pltpu.make_async_copy(colf.at[pl.ds(off, BE)], cbs[t], sins[t]).wait()
        pltpu.make_async_copy(normf.at[pl.ds(off, BE)], nbs[t],
                              sins[t]).wait()

    def _proc(t):
        def group(q, carry):
            p = q * L
            r = rbs[t][pl.ds(p, L)]
            c = cbs[t][pl.ds(p, L)]
            nv = nbs[t][pl.ds(p, L)]
            r5 = lax.shift_left(r, 2) + r
            for ch in range(IN_CH):
                xv = plsc.load_gather(xl, [r5 + ch])
                q2 = c + ch * N_PAD
                plsc.addupdate_scatter(
                    aggl, [lax.shift_right_logical(q2, 7),
                           lax.bitwise_and(q2, 127)], nv * xv)
            return carry

        lax.fori_loop(0, BE // L, group, 0)

    _in_start(0, 0)

    @pl.loop(0, MB, step=2)
    def _(g):
        _in_wait(g, 0)
        _in_start(g + 1, 1)
        _proc(0)
        _in_wait(g + 1, 1)

        @pl.when(g + 2 < MB)
        def _():
            _in_start(g + 2, 0)

        _proc(1)

    for f in range(4):
        pltpu.sync_copy(aggl.at[pl.ds(f * 100, 100)],
                        agg_sh.at[ridxb.at[f]], add=True)
    plsc.subcore_barrier()
    for j in range(3):
        pltpu.sync_copy(agg_sh.at[pl.ds((sid + 16 * j) * 8, 8)],
                        aggp.at[cid, pl.ds((sid + 16 * j) * 8, 8)])

    @pl.when(sid < 2)
    def _():
        pltpu.sync_copy(agg_sh.at[pl.ds((sid + 48) * 8, 8)],
                        aggp.at[cid, pl.ds((sid + 48) * 8, 8)])


BN = 1024
NSTEPS = N_PAD // BN


def _tc_body(aggp_ref, xT_ref, disT_ref, batT_ref, Dp_ref,
             W1_ref, b1T_ref, W2_ref, b2_ref, out_ref, cnt_ref):
    i = pl.program_id(0)

    @pl.when(i == 0)
    def _():
        out_ref[...] = jnp.zeros_like(out_ref)
        cnt_ref[...] = jnp.zeros_like(cnt_ref)

    a = aggp_ref[...]
    d = disT_ref[...]
    agg = a[0] + a[1] + d * d * xT_ref[...]
    hT = jax.lax.dot_general(W1_ref[...], agg, (((0,), (0,)), ((), ())),
                             preferred_element_type=jnp.float32)
    hT = jnp.maximum(hT + b1T_ref[...], 0.0)
    tT = jax.lax.dot_general(W2_ref[...], hT, (((0,), (0,)), ((), ())),
                             preferred_element_type=jnp.float32)
    Dm = Dp_ref[...]
    Dblk = Dm[0] + Dm[1]
    out_ref[...] += jax.lax.dot_general(
        Dblk, tT, (((1,), (1,)), ((), ())),
        preferred_element_type=jnp.float32)
    bt = batT_ref[...]
    gids = jax.lax.broadcasted_iota(jnp.int32, (G, BN), 0)
    oh = (bt == gids).astype(jnp.float32)
    cnt_ref[...] += jnp.sum(oh, axis=1, keepdims=True)

    @pl.when(i == NSTEPS - 1)
    def _():
        cnt = cnt_ref[...]
        inv = 1.0 / jnp.maximum(cnt, 1.0)
        msk = (cnt > 0.0).astype(jnp.float32)
        out_ref[...] = out_ref[...] * inv + b2_ref[...] * msk


_tc_call = pl.pallas_call(
    _tc_body,
    grid=(NSTEPS,),
    in_specs=[
        pl.BlockSpec((NC, IN_CH, BN), lambda i: (0, 0, i)),
        pl.BlockSpec((IN_CH, BN), lambda i: (0, i)),
        pl.BlockSpec((1, BN), lambda i: (0, i)),
        pl.BlockSpec((1, BN), lambda i: (0, i)),
        pl.BlockSpec((NC, G, BN), lambda i: (0, 0, i)),
        pl.BlockSpec((IN_CH, HID), lambda i: (0, 0)),
        pl.BlockSpec((HID, 1), lambda i: (0, 0)),
        pl.BlockSpec((HID, OUT), lambda i: (0, 0)),
        pl.BlockSpec((1, OUT), lambda i: (0, 0)),
    ],
    out_specs=pl.BlockSpec((G, OUT), lambda i: (0, 0)),
    out_shape=jax.ShapeDtypeStruct((G, OUT), jnp.float32),
    scratch_shapes=[pltpu.VMEM((G, 1), jnp.float32)],
)


def kernel(x, edge_index, edge_weight, batch, W1, b1, W2, b2):
    row = edge_index[0].astype(jnp.int32)
    col = edge_index[1].astype(jnp.int32)
    w = edge_weight.astype(jnp.float32)
    bi = batch.astype(jnp.int32)

    rowf = jnp.zeros((E_PAD,), jnp.int32).at[:E].set(row)
    colf = jnp.zeros((E_PAD,), jnp.int32).at[:E].set(col)
    wf = jnp.zeros((E_PAD,), jnp.float32).at[:E].set(w)

    batchf = jnp.zeros((N_PAD,), jnp.int32).at[:N].set(bi)
    xpad = (jnp.zeros((X_W,), jnp.float32)
            .at[: N * IN_CH].set(x.astype(jnp.float32).reshape(-1)))

    ridx4 = jnp.arange(400, dtype=jnp.int32).reshape(4, 100)

    disf, normf, Dp = _sc_main(rowf, colf, wf, batchf)
    aggp = _sc_agg(rowf, colf, normf, xpad, ridx4)

    aggp_r = aggp.reshape(NC, IN_CH, N_PAD)
    Dp_r = Dp.reshape(NC, G, N_PAD)
    xT5 = (jnp.zeros((IN_CH, N_PAD), jnp.float32)
           .at[:, :N].set(x.astype(jnp.float32).T))
    disT = disf.reshape(1, N_PAD)
    batT = jnp.full((1, N_PAD), G, jnp.int32).at[0, :N].set(bi)

    out = _tc_call(aggp_r, xT5, disT, batT, Dp_r,
                   W1.astype(jnp.float32), b1.reshape(HID, 1),
                   W2.astype(jnp.float32), b2.reshape(1, OUT))
    return out


# speedup vs baseline: 1.0094x; 1.0094x over previous
import functools

import jax
import jax.numpy as jnp
from jax import lax
from jax.experimental import pallas as pl
from jax.experimental.pallas import tpu as pltpu
from jax.experimental.pallas import tpu_sc as plsc

N = 10000
E = 160000
G = 64
IN_CH = 5
HID = 512
OUT = 256

NC = 2
NS = 16
NW = NC * NS
L = 16

N_PAD = 10240
NPW = N_PAD // NW
NPT = N_PAD // NS
E_PAD = 163840
EPW = E_PAD // NW
EPT = E_PAD // NS
BE = 512
SUBC = BE // 128
MB = EPW // BE
DB = EPT // BE
AGG_W = IN_CH * N_PAD
AGG_R = AGG_W // 128
X_W = 50048
D_ROWS = G * N_PAD
D_PT = D_ROWS // NS
ZB = 2048

_mesh = plsc.VectorSubcoreMesh(core_axis_name="c", subcore_axis_name="s")
_params = pltpu.CompilerParams(needs_layout_passes=False)


@functools.partial(
    pl.kernel,
    out_type=(
        jax.ShapeDtypeStruct((N_PAD,), jnp.float32),
        jax.ShapeDtypeStruct((E_PAD,), jnp.float32),
        jax.ShapeDtypeStruct((NC * D_ROWS,), jnp.float32),
    ),
    mesh=_mesh,
    compiler_params=_params,
    scratch_types=[
        pltpu.VMEM((N_PAD,), jnp.float32),
        pltpu.VMEM((N_PAD,), jnp.float32),
        pltpu.VMEM((N_PAD,), jnp.int32),
        pltpu.VMEM((BE,), jnp.int32),
        pltpu.VMEM((BE,), jnp.int32),
        pltpu.VMEM((BE,), jnp.float32),
        pltpu.VMEM((BE,), jnp.float32),
        pltpu.VMEM((BE,), jnp.int32),
        pltpu.VMEM((BE,), jnp.int32),
        pltpu.VMEM((1, 128), jnp.int32),
        pltpu.VMEM((1, 128), jnp.int32),
        pltpu.VMEM((1, 128), jnp.int32),
        pltpu.VMEM((1, 128), jnp.int32),
        pltpu.VMEM((BE,), jnp.float32),
        pltpu.VMEM((128,), jnp.float32),
        pltpu.SemaphoreType.DMA,
        pltpu.VMEM((NPT,), jnp.float32),
        pltpu.VMEM((NPT,), jnp.float32),
        pltpu.VMEM((ZB,), jnp.float32),
        pltpu.SemaphoreType.DMA,
        pltpu.SemaphoreType.DMA,
        pltpu.VMEM_SHARED((N_PAD,), jnp.float32),
        pltpu.VMEM_SHARED((N_PAD,), jnp.int32),
        pltpu.VMEM_SHARED((D_ROWS,), jnp.float32),
        pltpu.VMEM_SHARED((NS * N_PAD,), jnp.float32),
    ],
)
def _sc_main(rowf, colf, wf, batchf, disf, normf, Dp,
             degl, disl, batchl, cb0, cb1, wb0, wb1, rb0, rb1,
             didx0, didx1, didx2, didx3, nva, nv0, sfl, accb, tmp0, zb,
             sin0, sin1,
             dis_sh, batch_sh, D_sh, slots):
    cid = lax.axis_index("c")
    sid = lax.axis_index("s")
    wid = cid * NS + sid
    iota = lax.iota(jnp.int32, L)
    z16 = jnp.zeros((L,), jnp.int32)
    zf16 = jnp.zeros((L,), jnp.float32)
    cbs, wbs, rbs = (cb0, cb1), (wb0, wb1), (rb0, rb1)
    sins = (sin0, sin1)

    @pl.loop(0, ZB // L)
    def _(q):
        zb[pl.ds(q * L, L)] = zf16

    @pl.loop(0, D_PT // ZB)
    def _(q):
        pltpu.sync_copy(zb, D_sh.at[pl.ds(sid * D_PT + q * ZB, ZB)])

    @pl.loop(0, N_PAD // L)
    def _(q):
        degl[pl.ds(q * L, L)] = zf16

    @pl.when(sid == 0)
    def _():
        pltpu.sync_copy(batchf, batch_sh)

    dbase = sid * EPT

    def _deg_start(g, t):
        off = dbase + g * BE
        pltpu.async_copy(colf.at[pl.ds(off, BE)], cbs[t], sins[t])
        pltpu.async_copy(wf.at[pl.ds(off, BE)], wbs[t], sins[t])

    def _deg_wait(g, t):
        off = dbase + g * BE
        pltpu.make_async_copy(colf.at[pl.ds(off, BE)], cbs[t], sins[t]).wait()
        pltpu.make_async_copy(wf.at[pl.ds(off, BE)], wbs[t], sins[t]).wait()

    def _deg_proc(t):
        def group(q, carry):
            p = q * L
            ci = cbs[t][pl.ds(p, L)]
            wv = wbs[t][pl.ds(p, L)]
            plsc.addupdate_scatter(degl, [ci], wv)
            return carry

        lax.fori_loop(0, BE // L, group, 0, unroll=2)

    _deg_start(0, 0)

    @pl.loop(0, DB, step=2)
    def _(g):
        _deg_wait(g, 0)
        _deg_start(g + 1, 1)
        _deg_proc(0)
        _deg_wait(g + 1, 1)

        @pl.when(g + 2 < DB)
        def _():
            _deg_start(g + 2, 0)

        _deg_proc(1)

    pltpu.sync_copy(degl, slots.at[pl.ds(sid * N_PAD, N_PAD)])
    plsc.subcore_barrier()
    nb0 = sid * NPT
    pltpu.sync_copy(slots.at[pl.ds(nb0, NPT)], accb)
    for k in range(1, NS):
        pltpu.sync_copy(slots.at[pl.ds(k * N_PAD + nb0, NPT)], tmp0)

        def addg(q, carry):
            sl = pl.ds(q * L, L)
            accb[sl] = accb[sl] + tmp0[sl]
            return carry

        lax.fori_loop(0, NPT // L, addg, 0)

    for gg in range(NPT // L):
        sl = pl.ds(gg * L, L)
        d = accb[sl] + 1.0
        i32 = plsc.bitcast(d, jnp.int32)
        seed = jnp.int32(0x5F3759DF) - lax.shift_right_arithmetic(i32, 1)
        y = plsc.bitcast(seed, jnp.float32)
        for _ in range(3):
            y = y * (1.5 - 0.5 * d * y * y)
        accb[sl] = y
    pltpu.sync_copy(accb, dis_sh.at[pl.ds(nb0, NPT)])

    @pl.when(cid == 0)
    def _():
        pltpu.sync_copy(accb, disf.at[pl.ds(nb0, NPT)])

    plsc.subcore_barrier()
    pltpu.sync_copy(dis_sh, disl)
    pltpu.sync_copy(batch_sh, batchl)

    ebase = wid * EPW

    def _in_start(g, t):
        off = ebase + g * BE
        pltpu.async_copy(rowf.at[pl.ds(off, BE)], rbs[t], sins[t])
        pltpu.async_copy(colf.at[pl.ds(off, BE)], cbs[t], sins[t])
        pltpu.async_copy(wf.at[pl.ds(off, BE)], wbs[t], sins[t])

    def _in_wait(g, t):
        off = ebase + g * BE
        pltpu.make_async_copy(rowf.at[pl.ds(off, BE)], rbs[t], sins[t]).wait()
        pltpu.make_async_copy(colf.at[pl.ds(off, BE)], cbs[t], sins[t]).wait()
        pltpu.make_async_copy(wf.at[pl.ds(off, BE)], wbs[t],
                              sins[t]).wait()

    didxs = (didx0, didx1, didx2, didx3)

    def _proc_burst(g, t):
        descs = []
        for sc_i in range(SUBC):
            for k in range(8):
                p = sc_i * 128 + k * L
                sl = pl.ds(p, L)
                r = rbs[t][sl]
                c = cbs[t][sl]
                wv = wbs[t][sl]
                dr = plsc.load_gather(disl, [r])
                dc = plsc.load_gather(disl, [c])
                nv = dr * wv * dc
                gb = plsc.load_gather(batchl, [c])
                di = gb * N_PAD + r
                didxs[sc_i][0, pl.ds(k * L, L)] = di
                nva[pl.ds(p, L)] = nv
            pltpu.sync_copy(nva.at[pl.ds(sc_i * 128, 128)],
                            D_sh.at[didxs[sc_i].at[0]], add=True)
        descs.append(pltpu.async_copy(
            nva, normf.at[pl.ds(ebase + g * BE, BE)], sfl))
        for dsc in descs:
            dsc.wait()

    _in_start(0, 0)

    @pl.loop(0, MB, step=2)
    def _(g):
        _in_wait(g, 0)
        _in_start(g + 1, 1)
        _proc_burst(g, 0)
        _in_wait(g + 1, 1)

        @pl.when(g + 2 < MB)
        def _():
            _in_start(g + 2, 0)

        _proc_burst(g + 1, 1)

    sbase = wid * NPW
    for t in range(3):
        for k in range(8):
            gq = t * 8 + k
            if gq < NPW // L:
                n16 = sbase + gq * L + iota
                dn = plsc.load_gather(disl, [n16])
                v2 = jnp.where(n16 < N, dn * dn, 0.0)
                gb = plsc.load_gather(batchl, [n16])
                di = gb * N_PAD + n16
                didx0[0, pl.ds(k * L, L)] = di
                nv0[pl.ds(k * L, L)] = v2
            else:
                didx0[0, pl.ds(k * L, L)] = z16
                nv0[pl.ds(k * L, L)] = zf16
        pltpu.sync_copy(nv0, D_sh.at[didx0.at[0]], add=True)

    plsc.subcore_barrier()
    pltpu.sync_copy(D_sh.at[pl.ds(sid * D_PT, D_PT)],
                    Dp.at[pl.ds(cid * D_ROWS + sid * D_PT, D_PT)])


@functools.partial(
    pl.kernel,
    out_type=jax.ShapeDtypeStruct((NC, AGG_R, 128), jnp.float32),
    mesh=_mesh,
    compiler_params=_params,
    scratch_types=[
        pltpu.VMEM((AGG_R, 128), jnp.float32),
        pltpu.VMEM((X_W,), jnp.float32),
        pltpu.VMEM((BE,), jnp.int32),
        pltpu.VMEM((BE,), jnp.int32),
        pltpu.VMEM((BE,), jnp.int32),
        pltpu.VMEM((BE,), jnp.int32),
        pltpu.VMEM((BE,), jnp.float32),
        pltpu.VMEM((BE,), jnp.float32),
        pltpu.VMEM((4, 100), jnp.int32),
        pltpu.VMEM((8, 128), jnp.float32),
        pltpu.SemaphoreType.DMA,
        pltpu.SemaphoreType.DMA,
        pltpu.VMEM_SHARED((AGG_R, 128), jnp.float32),
        pltpu.VMEM_SHARED((X_W,), jnp.float32),
    ],
)
def _sc_agg(rowf, colf, normf, xpad, ridx4, aggp,
            aggl, xl, rb0, rb1, cb0, cb1, nb0, nb1, ridxb, zb,
            sin0, sin1, agg_sh, x_sh):
    cid = lax.axis_index("c")
    sid = lax.axis_index("s")
    wid = cid * NS + sid
    zf16 = jnp.zeros((L,), jnp.float32)
    rbs, cbs, nbs = (rb0, rb1), (cb0, cb1), (nb0, nb1)
    sins = (sin0, sin1)
    pltpu.sync_copy(ridx4, ridxb)
    for rr in range(8):
        for c8 in range(8):
            zb[rr, pl.ds(c8 * L, L)] = zf16

    @pl.when(sid == 0)
    def _():
        pltpu.sync_copy(xpad, x_sh)

    for j in range(3):
        pltpu.sync_copy(zb, agg_sh.at[pl.ds((sid + 16 * j) * 8, 8)])

    @pl.when(sid < 2)
    def _():
        pltpu.sync_copy(zb, agg_sh.at[pl.ds((sid + 48) * 8, 8)])

    plsc.subcore_barrier()
    pltpu.sync_copy(x_sh, xl)
    pltpu.sync_copy(agg_sh, aggl)
    plsc.subcore_barrier()
    ebase = wid * EPW

    def _in_start(g, t):
        off = ebase + g * BE
        pltpu.async_copy(rowf.at[pl.ds(off, BE)], rbs[t], sins[t])
        pltpu.async_copy(colf.at[pl.ds(off, BE)], cbs[t], sins[t])
        pltpu.async_copy(normf.at[pl.ds(off, BE)], nbs[t], sins[t])

    def _in_wait(g, t):
        off = ebase + g * BE
        pltpu.make_async_copy(rowf.at[pl.ds(off, BE)], rbs[t], sins[t]).wait()
        pltpu.make_async_copy(colf.at[pl.ds(off, BE)], cbs[t], sins[t]).wait()
        pltpu.make_async_copy(normf.at[pl.ds(off, BE)], nbs[t],
                              sins[t]).wait()

    def _proc(t):
        def group(q, carry):
            p = q * L
            r = rbs[t][pl.ds(p, L)]
            c = cbs[t][pl.ds(p, L)]
            nv = nbs[t][pl.ds(p, L)]
            r5 = lax.shift_left(r, 2) + r
            for ch in range(IN_CH):
                xv = plsc.load_gather(xl, [r5 + ch])
                q2 = c + ch * N_PAD
                plsc.addupdate_scatter(
                    aggl, [lax.shift_right_logical(q2, 7),
                           lax.bitwise_and(q2, 127)], nv * xv)
            return carry

        lax.fori_loop(0, BE // L, group, 0, unroll=2)

    _in_start(0, 0)

    @pl.loop(0, MB, step=2)
    def _(g):
        _in_wait(g, 0)
        _in_start(g + 1, 1)
        _proc(0)
        _in_wait(g + 1, 1)

        @pl.when(g + 2 < MB)
        def _():
            _in_start(g + 2, 0)

        _proc(1)

    for f in range(4):
        pltpu.sync_copy(aggl.at[pl.ds(f * 100, 100)],
                        agg_sh.at[ridxb.at[f]], add=True)
    plsc.subcore_barrier()
    for j in range(3):
        pltpu.sync_copy(agg_sh.at[pl.ds((sid + 16 * j) * 8, 8)],
                        aggp.at[cid, pl.ds((sid + 16 * j) * 8, 8)])

    @pl.when(sid < 2)
    def _():
        pltpu.sync_copy(agg_sh.at[pl.ds((sid + 48) * 8, 8)],
                        aggp.at[cid, pl.ds((sid + 48) * 8, 8)])


BN = 1024
NSTEPS = N_PAD // BN


def _tc_body(aggp_ref, xT_ref, disT_ref, batT_ref, Dp_ref,
             W1_ref, b1T_ref, W2_ref, b2_ref, out_ref, cnt_ref):
    i = pl.program_id(0)

    @pl.when(i == 0)
    def _():
        out_ref[...] = jnp.zeros_like(out_ref)
        cnt_ref[...] = jnp.zeros_like(cnt_ref)

    a = aggp_ref[...]
    d = disT_ref[...]
    agg = a[0] + a[1] + d * d * xT_ref[...]
    hT = jax.lax.dot_general(W1_ref[...], agg, (((0,), (0,)), ((), ())),
                             preferred_element_type=jnp.float32)
    hT = jnp.maximum(hT + b1T_ref[...], 0.0)
    tT = jax.lax.dot_general(W2_ref[...], hT, (((0,), (0,)), ((), ())),
                             preferred_element_type=jnp.float32)
    Dm = Dp_ref[...]
    Dblk = Dm[0] + Dm[1]
    out_ref[...] += jax.lax.dot_general(
        Dblk, tT, (((1,), (1,)), ((), ())),
        preferred_element_type=jnp.float32)
    bt = batT_ref[...]
    gids = jax.lax.broadcasted_iota(jnp.int32, (G, BN), 0)
    oh = (bt == gids).astype(jnp.float32)
    cnt_ref[...] += jnp.sum(oh, axis=1, keepdims=True)

    @pl.when(i == NSTEPS - 1)
    def _():
        cnt = cnt_ref[...]
        inv = 1.0 / jnp.maximum(cnt, 1.0)
        msk = (cnt > 0.0).astype(jnp.float32)
        out_ref[...] = out_ref[...] * inv + b2_ref[...] * msk


_tc_call = pl.pallas_call(
    _tc_body,
    grid=(NSTEPS,),
    in_specs=[
        pl.BlockSpec((NC, IN_CH, BN), lambda i: (0, 0, i)),
        pl.BlockSpec((IN_CH, BN), lambda i: (0, i)),
        pl.BlockSpec((1, BN), lambda i: (0, i)),
        pl.BlockSpec((1, BN), lambda i: (0, i)),
        pl.BlockSpec((NC, G, BN), lambda i: (0, 0, i)),
        pl.BlockSpec((IN_CH, HID), lambda i: (0, 0)),
        pl.BlockSpec((HID, 1), lambda i: (0, 0)),
        pl.BlockSpec((HID, OUT), lambda i: (0, 0)),
        pl.BlockSpec((1, OUT), lambda i: (0, 0)),
    ],
    out_specs=pl.BlockSpec((G, OUT), lambda i: (0, 0)),
    out_shape=jax.ShapeDtypeStruct((G, OUT), jnp.float32),
    scratch_shapes=[pltpu.VMEM((G, 1), jnp.float32)],
)


def kernel(x, edge_index, edge_weight, batch, W1, b1, W2, b2):
    row = edge_index[0].astype(jnp.int32)
    col = edge_index[1].astype(jnp.int32)
    w = edge_weight.astype(jnp.float32)
    bi = batch.astype(jnp.int32)

    rowf = jnp.zeros((E_PAD,), jnp.int32).at[:E].set(row)
    colf = jnp.zeros((E_PAD,), jnp.int32).at[:E].set(col)
    wf = jnp.zeros((E_PAD,), jnp.float32).at[:E].set(w)

    batchf = jnp.zeros((N_PAD,), jnp.int32).at[:N].set(bi)
    xpad = (jnp.zeros((X_W,), jnp.float32)
            .at[: N * IN_CH].set(x.astype(jnp.float32).reshape(-1)))

    ridx4 = jnp.arange(400, dtype=jnp.int32).reshape(4, 100)

    disf, normf, Dp = _sc_main(rowf, colf, wf, batchf)
    aggp = _sc_agg(rowf, colf, normf, xpad, ridx4)

    aggp_r = aggp.reshape(NC, IN_CH, N_PAD)
    Dp_r = Dp.reshape(NC, G, N_PAD)
    xT5 = (jnp.zeros((IN_CH, N_PAD), jnp.float32)
           .at[:, :N].set(x.astype(jnp.float32).T))
    disT = disf.reshape(1, N_PAD)
    batT = jnp.full((1, N_PAD), G, jnp.int32).at[0, :N].set(bi)

    out = _tc_call(aggp_r, xT5, disT, batT, Dp_r,
                   W1.astype(jnp.float32), b1.reshape(HID, 1),
                   W2.astype(jnp.float32), b2.reshape(1, OUT))
    return out
